# baseline (speedup 1.0000x reference)
import jax
import jax.numpy as jnp
from jax import lax
from jax.experimental import pallas as pl
from jax.experimental.pallas import tpu as pltpu
from jax.experimental.pallas import tpu_sc as plsc

_VOCAB = 1000000
_DIM = 64
_B = 4096
_L = 200
_NC = 2
_NS = 16
_NW = _NC * _NS
_BLK = _B // _NW
_TP = 133


def _gather_kernel(ids_hbm, table_hbm, out_hbm,
                   raw0, raw1, idx0, idx1, off0, off1,
                   rows0, rows1, t0, t1,
                   gsem0, gsem1, wsem0, wsem1):
    raw = (raw0, raw1)
    idx = (idx0, idx1)
    off = (off0, off1)
    rows = (rows0, rows1)
    tblk = (t0, t1)
    gsem = (gsem0, gsem1)
    wsem = (wsem0, wsem1)

    wid = lax.axis_index("s") * _NC + lax.axis_index("c")
    b0 = wid * _BLK
    d_idx = [lax.iota(jnp.int32, 16) + 16 * dg for dg in range(4)]

    def load_idx(p, l):
        pltpu.sync_copy(ids_hbm.at[l, pl.ds(b0, _BLK)], raw[p])
        for g in range(8):
            v = raw[p][pl.ds(16 * g, 16)]
            idx[p][pl.ds(16 * g, 16)] = v >> 1

    def load_off(p):
        for g in range(8):
            v = raw[p][pl.ds(16 * g, 16)]
            off[p][pl.ds(16 * g, 16)] = (v & 1) << 6

    def transpose_unit(p):
        r, t, o = rows[p], tblk[p], off[p]

        @pl.loop(0, _BLK, unroll=4)
        def _(j):
            jv = jnp.full((16,), j, jnp.int32)
            ov = plsc.load_gather(o, [jv])
            for dg in range(4):
                x = plsc.load_gather(r, [jv, d_idx[dg] + ov])
                plsc.store_scatter(t, [d_idx[dg], jv], x)

    def start_gather(p):
        pltpu.async_copy(table_hbm.at[idx[p]], rows[p], gsem[p])

    def wait_gather(p):
        pltpu.make_async_copy(table_hbm.at[idx[p]], rows[p], gsem[p]).wait()

    def start_write(p, l):
        pltpu.async_copy(tblk[p].at[:, pl.ds(0, _BLK)],
                         out_hbm.at[l, :, pl.ds(b0, _BLK)], wsem[p])

    def wait_write(p, l):
        pltpu.make_async_copy(tblk[p].at[:, pl.ds(0, _BLK)],
                              out_hbm.at[l, :, pl.ds(b0, _BLK)], wsem[p]).wait()

    for p in (0, 1):
        load_idx(p, p)
        load_off(p)
        start_gather(p)
    for p in (0, 1):
        wait_gather(p)
        load_idx(p, p + 2)
        transpose_unit(p)
        load_off(p)
        start_gather(p)
        start_write(p, p)

    def step(k, carry):
        for p in (0, 1):
            l = 2 * k + p
            wait_gather(p)
            load_idx(p, l + 2)
            wait_write(p, l)
            transpose_unit(p)
            load_off(p)
            start_gather(p)
            start_write(p, l)
        return carry

    lax.fori_loop(1, (_L - 2) // 2, step, 0)

    for p in (0, 1):
        l = _L - 2 + p
        wait_gather(p)
        wait_write(p, l)
        transpose_unit(p)
        start_write(p, l)
    for p in (0, 1):
        wait_write(p, _L - 2 + p)


@jax.jit
def kernel(input_ids, table):
    table2 = table.reshape(_VOCAB // 2, 2 * _DIM)
    ids_t = input_ids.T
    mesh = plsc.VectorSubcoreMesh(
        core_axis_name="c", subcore_axis_name="s",
        num_cores=_NC, num_subcores=_NS,
    )
    out3 = pl.kernel(
        _gather_kernel,
        out_type=jax.ShapeDtypeStruct((_L, _DIM, _B), jnp.float32),
        mesh=mesh,
        scratch_types=(
            [pltpu.VMEM((_BLK,), jnp.int32) for _ in range(6)]
            + [pltpu.VMEM((_BLK, 2 * _DIM), jnp.float32) for _ in range(2)]
            + [pltpu.VMEM((_DIM, _TP), jnp.float32) for _ in range(2)]
            + [pltpu.SemaphoreType.DMA for _ in range(4)]
        ),
        compiler_params=pltpu.CompilerParams(needs_layout_passes=False),
    )(ids_t, table2)
    return out3.transpose(2, 0, 1)


# speedup vs baseline: 1.5684x; 1.5684x over previous
import jax
import jax.numpy as jnp
from jax import lax
from jax.experimental import pallas as pl
from jax.experimental.pallas import tpu as pltpu
from jax.experimental.pallas import tpu_sc as plsc

_VOCAB = 1000000
_DIM = 64
_B = 4096
_L = 200
_TOTAL = _B * _L
_NC = 2
_NS = 16
_NW = _NC * _NS
_PER_W = _TOTAL // _NW
_CHUNK = 400
_NCHUNK = _PER_W // _CHUNK
_NBUF = 4


def _gather_kernel(ids_hbm, table_hbm, out_hbm, *scratch):
    idx = scratch[0:_NBUF]
    rows = scratch[_NBUF:2 * _NBUF]
    gsem = scratch[2 * _NBUF:3 * _NBUF]
    wsem = scratch[3 * _NBUF:4 * _NBUF]

    wid = lax.axis_index("s") * _NC + lax.axis_index("c")
    base = wid * _PER_W

    for b in range(_NBUF):
        pltpu.sync_copy(ids_hbm.at[pl.ds(base + b * _CHUNK, _CHUNK)], idx[b])
        pltpu.async_copy(table_hbm.at[idx[b]], rows[b], gsem[b])

    def step(k, carry):
        c = k * _NBUF
        for b in range(_NBUF):
            off = base + (c + b) * _CHUNK
            pltpu.make_async_copy(table_hbm.at[idx[b]], rows[b], gsem[b]).wait()
            pltpu.async_copy(rows[b], out_hbm.at[pl.ds(off, _CHUNK)], wsem[b])
        for b in range(_NBUF):
            off = base + (c + _NBUF + b) * _CHUNK
            pltpu.make_async_copy(
                rows[b], out_hbm.at[pl.ds(off, _CHUNK)], wsem[b]).wait()
            pltpu.sync_copy(ids_hbm.at[pl.ds(off, _CHUNK)], idx[b])
            pltpu.async_copy(table_hbm.at[idx[b]], rows[b], gsem[b])
        return carry

    lax.fori_loop(0, (_NCHUNK - 2 * _NBUF) // _NBUF + 1, step, 0)

    for b in range(_NBUF):
        off = base + (_NCHUNK - _NBUF + b) * _CHUNK
        pltpu.make_async_copy(table_hbm.at[idx[b]], rows[b], gsem[b]).wait()
        pltpu.async_copy(rows[b], out_hbm.at[pl.ds(off, _CHUNK)], wsem[b])
    for b in range(_NBUF):
        off = base + (_NCHUNK - _NBUF + b) * _CHUNK
        pltpu.make_async_copy(
            rows[b], out_hbm.at[pl.ds(off, _CHUNK)], wsem[b]).wait()


@jax.jit
def kernel(input_ids, table):
    ids_flat = input_ids.reshape(_TOTAL)
    mesh = plsc.VectorSubcoreMesh(
        core_axis_name="c", subcore_axis_name="s",
        num_cores=_NC, num_subcores=_NS,
    )
    out = pl.kernel(
        _gather_kernel,
        out_type=jax.ShapeDtypeStruct((_TOTAL, _DIM), jnp.float32),
        mesh=mesh,
        scratch_types=(
            [pltpu.VMEM((_CHUNK,), jnp.int32) for _ in range(_NBUF)]
            + [pltpu.VMEM((_CHUNK, _DIM), jnp.float32) for _ in range(_NBUF)]
            + [pltpu.SemaphoreType.DMA for _ in range(2 * _NBUF)]
        ),
        compiler_params=pltpu.CompilerParams(use_tc_tiling_on_sc=False),
    )(ids_flat, table)
    return out.reshape(_B, _L, _DIM)
